# baseline (speedup 1.0000x reference)
import functools

import jax
import jax.numpy as jnp
from jax import lax
from jax.experimental import pallas as pl
from jax.experimental.pallas import tpu as pltpu
from jax.experimental.pallas import tpu_sc as plsc

N = 10000
E = 320000
D = 128
H = 128
C = 10
G = 64

NC = 2
NS = 16
NW = NC * NS
EPW = E // NW
K = 80
CH = EPW // K
NT = 5
TCH = CH // NT
NPAD = 10240
RPT = NPAD // NS

_mesh = plsc.VectorSubcoreMesh(core_axis_name="c", subcore_axis_name="s",
                               num_cores=NC, num_subcores=NS)


def _deg_body(dst_hbm, ones_hbm, zeros_hbm, deg_out, dstloc, ones_v, acc):
    c = lax.axis_index("c")
    s = lax.axis_index("s")
    wid = s * NC + c
    pltpu.sync_copy(zeros_hbm, acc.at[pl.ds(s * RPT, RPT)])
    pltpu.sync_copy(ones_hbm, ones_v)
    pltpu.sync_copy(dst_hbm.at[wid], dstloc)
    plsc.subcore_barrier()

    def chunk(g, carry):
        pltpu.sync_copy(ones_v, acc.at[dstloc.at[g]], add=True)
        return carry

    lax.fori_loop(0, CH, chunk, 0)
    plsc.subcore_barrier()
    pltpu.sync_copy(acc.at[pl.ds(s * RPT, RPT)],
                    deg_out.at[c, pl.ds(s * RPT, RPT)])


_deg_call = pl.kernel(
    _deg_body,
    out_type=jax.ShapeDtypeStruct((NC, NPAD, H), jnp.float32),
    mesh=_mesh,
    scratch_types=[
        pltpu.VMEM((CH, K), jnp.int32),
        pltpu.VMEM((K, H), jnp.float32),
        pltpu.VMEM_SHARED((NPAD, H), jnp.float32),
    ],
)


NBUF = 4
TAILR = N - (NS - 1) * RPT


def _agg_body(p_hbm, ed_hbm, zeros_hbm, out_hbm,
              ed0, rows0, rows1, rows2, rows3, acc,
              seme0, semg0, semg1, semg2, semg3,
              sems0, sems1, sems2, sems3):
    c = lax.axis_index("c")
    s = lax.axis_index("s")
    wid = s * NC + c

    @pl.when(s < NS - 1)
    def _():
        pltpu.sync_copy(zeros_hbm, acc.at[pl.ds(s * RPT, RPT)])

    @pl.when(s == NS - 1)
    def _():
        pltpu.sync_copy(zeros_hbm.at[pl.ds(0, TAILR)],
                        acc.at[pl.ds((NS - 1) * RPT, TAILR)])

    rows = (rows0, rows1, rows2, rows3)
    semg = (semg0, semg1, semg2, semg3)
    sems = (sems0, sems1, sems2, sems3)
    pltpu.async_copy(ed_hbm.at[wid, 0], ed0, seme0)
    plsc.subcore_barrier()

    for t in range(NT):
        eb = ed0
        pltpu.make_async_copy(ed_hbm.at[wid, t], eb, seme0).wait()
        pltpu.async_copy(p_hbm.at[eb.at[0, 0]], rows[0], semg[0])
        pltpu.async_copy(p_hbm.at[eb.at[0, 1]], rows[1], semg[1])
        pltpu.async_copy(p_hbm.at[eb.at[0, 2]], rows[2], semg[2])

        def step(i, carry, eb=eb):
            for b in range(NBUF):
                j = NBUF * i + b
                nb = (b + NBUF - 1) % NBUF

                @pl.when(j + NBUF - 1 < TCH)
                def _():
                    @pl.when(j >= 1)
                    def _():
                        pltpu.make_async_copy(
                            rows[nb], acc.at[eb.at[1, 0]], sems[nb]).wait()
                    pltpu.async_copy(p_hbm.at[eb.at[0, j + NBUF - 1]],
                                     rows[nb], semg[nb])

                pltpu.make_async_copy(p_hbm.at[eb.at[0, j]],
                                      rows[b], semg[b]).wait()
                pltpu.async_copy(rows[b], acc.at[eb.at[1, j]],
                                 sems[b], add=True)
            return carry

        lax.fori_loop(0, TCH // NBUF, step, 0)
        j = TCH - 1
        pltpu.make_async_copy(p_hbm.at[eb.at[0, j]],
                              rows[j % NBUF], semg[j % NBUF]).wait()
        pltpu.async_copy(rows[j % NBUF], acc.at[eb.at[1, j]],
                         sems[j % NBUF], add=True)
        for b in range(NBUF):
            pltpu.make_async_copy(rows[b], acc.at[eb.at[1, 0]],
                                  sems[b]).wait()
        if t + 1 < NT:
            pltpu.async_copy(ed_hbm.at[wid, t + 1], ed0, seme0)

    plsc.subcore_barrier()

    @pl.when(s < NS - 1)
    def _():
        pltpu.sync_copy(acc.at[pl.ds(s * RPT, RPT)],
                        out_hbm.at[c, pl.ds(s * RPT, RPT)])

    @pl.when(s == NS - 1)
    def _():
        pltpu.sync_copy(acc.at[pl.ds((NS - 1) * RPT, TAILR)],
                        out_hbm.at[c, pl.ds((NS - 1) * RPT, TAILR)])


_agg_call = pl.kernel(
    _agg_body,
    out_type=jax.ShapeDtypeStruct((NC, N, H), jnp.float32),
    mesh=_mesh,
    scratch_types=[
        pltpu.VMEM((2, TCH, K), jnp.int32),
        pltpu.VMEM((K, H), jnp.float32),
        pltpu.VMEM((K, H), jnp.float32),
        pltpu.VMEM((K, H), jnp.float32),
        pltpu.VMEM((K, H), jnp.float32),
        pltpu.VMEM_SHARED((N, H), jnp.float32),
        pltpu.SemaphoreType.DMA,
        pltpu.SemaphoreType.DMA,
        pltpu.SemaphoreType.DMA,
        pltpu.SemaphoreType.DMA,
        pltpu.SemaphoreType.DMA,
        pltpu.SemaphoreType.DMA,
        pltpu.SemaphoreType.DMA,
        pltpu.SemaphoreType.DMA,
        pltpu.SemaphoreType.DMA,
    ],
)


RB = 2000
NBLK = N // RB


def _prep_body(d0_ref, d1_ref, x_ref, w_ref, p_ref, dinv_ref):
    deg = d0_ref[:, 0:1] + d1_ref[:, 0:1] + 1.0
    dinv = lax.rsqrt(deg)
    dinv_ref[...] = dinv
    p_ref[...] = jnp.dot(x_ref[...] * dinv, w_ref[...],
                         preferred_element_type=jnp.float32)


def _prep_call(d0, d1, x, w):
    return pl.pallas_call(
        _prep_body,
        grid=(NBLK,),
        in_specs=[
            pl.BlockSpec((RB, H), lambda r: (r, 0)),
            pl.BlockSpec((RB, H), lambda r: (r, 0)),
            pl.BlockSpec((RB, D), lambda r: (r, 0)),
            pl.BlockSpec((D, H), lambda r: (0, 0)),
        ],
        out_specs=[
            pl.BlockSpec((RB, H), lambda r: (r, 0)),
            pl.BlockSpec((RB, 1), lambda r: (r, 0)),
        ],
        out_shape=[
            jax.ShapeDtypeStruct((N, H), jnp.float32),
            jax.ShapeDtypeStruct((N, 1), jnp.float32),
        ],
    )(d0, d1, x, w)


def _layer_body(acc_ref, p_ref, dinv_ref, b_ref, w_ref, pn_ref):
    acc = acc_ref[0] + acc_ref[1]
    dinv = dinv_ref[...]
    a = jnp.maximum(dinv * (acc + p_ref[...]) + b_ref[...], 0.0)
    pn_ref[...] = jnp.dot(a * dinv, w_ref[...],
                          preferred_element_type=jnp.float32)


def _layer_call(accp, p, dinv, b, w):
    return pl.pallas_call(
        _layer_body,
        grid=(NBLK,),
        in_specs=[
            pl.BlockSpec((NC, RB, H), lambda r: (0, r, 0)),
            pl.BlockSpec((RB, H), lambda r: (r, 0)),
            pl.BlockSpec((RB, 1), lambda r: (r, 0)),
            pl.BlockSpec((1, H), lambda r: (0, 0)),
            pl.BlockSpec((H, H), lambda r: (0, 0)),
        ],
        out_specs=pl.BlockSpec((RB, H), lambda r: (r, 0)),
        out_shape=jax.ShapeDtypeStruct((N, H), jnp.float32),
    )(accp, p, dinv, b, w)


def _final_body(acc_ref, p_ref, dinv_ref, b_ref, batch_ref,
                l1w_ref, l1b_ref, l2w_ref, l2b_ref, out_ref, psum, cnt):
    r = pl.program_id(0)

    @pl.when(r == 0)
    def _():
        psum[...] = jnp.zeros_like(psum)
        cnt[...] = jnp.zeros_like(cnt)

    acc = acc_ref[0] + acc_ref[1]
    h = jnp.maximum(dinv_ref[...] * (acc + p_ref[...]) + b_ref[...], 0.0)
    gids = lax.broadcasted_iota(jnp.int32, (G, RB), 0)
    onehot = (gids == batch_ref[0]).astype(jnp.float32)
    psum[...] += jnp.dot(onehot, h, preferred_element_type=jnp.float32)
    cnt[...] += jnp.sum(onehot, axis=1, keepdims=True)

    @pl.when(r == NBLK - 1)
    def _():
        pooled = psum[...] / jnp.maximum(cnt[...], 1.0)
        z = jnp.maximum(
            jnp.dot(pooled, l1w_ref[...], preferred_element_type=jnp.float32)
            + l1b_ref[...], 0.0)
        logits = (jnp.dot(z, l2w_ref[...], preferred_element_type=jnp.float32)
                  + l2b_ref[...])
        m = jnp.max(logits, axis=1, keepdims=True)
        lse = jnp.log(jnp.sum(jnp.exp(logits - m), axis=1, keepdims=True)) + m
        out_ref[...] = logits - lse


def _final_call(accp, p, dinv, b, batch2, l1w, l1b, l2w, l2b):
    return pl.pallas_call(
        _final_body,
        grid=(NBLK,),
        in_specs=[
            pl.BlockSpec((NC, RB, H), lambda r: (0, r, 0)),
            pl.BlockSpec((RB, H), lambda r: (r, 0)),
            pl.BlockSpec((RB, 1), lambda r: (r, 0)),
            pl.BlockSpec((1, H), lambda r: (0, 0)),
            pl.BlockSpec((1, 1, RB), lambda r: (r, 0, 0)),
            pl.BlockSpec((H, H), lambda r: (0, 0)),
            pl.BlockSpec((1, H), lambda r: (0, 0)),
            pl.BlockSpec((H, C), lambda r: (0, 0)),
            pl.BlockSpec((1, C), lambda r: (0, 0)),
        ],
        out_specs=pl.BlockSpec((G, C), lambda r: (0, 0)),
        out_shape=jax.ShapeDtypeStruct((G, C), jnp.float32),
        scratch_shapes=[
            pltpu.VMEM((G, H), jnp.float32),
            pltpu.VMEM((G, 1), jnp.float32),
        ],
    )(accp, p, dinv, b, batch2, l1w, l1b, l2w, l2b)


def kernel(x, edge_index, batch, W1, b1, W2, b2, W3, b3, W4, b4,
           lin1_W, lin1_b, lin2_W, lin2_b):
    src2 = edge_index[0].reshape(NW, CH, K)
    dst2 = edge_index[1].reshape(NW, CH, K)
    ed = jnp.stack([edge_index[0].reshape(NW, NT, TCH, K),
                    edge_index[1].reshape(NW, NT, TCH, K)], axis=2)
    ones128 = jnp.ones((K, H), jnp.float32)
    zer128 = jnp.zeros((RPT, H), jnp.float32)

    degp = _deg_call(dst2, ones128, zer128)

    p, dinv = _prep_call(degp[0], degp[1], x, W1)
    for b, w in ((b1, W2), (b2, W3), (b3, W4)):
        accp = _agg_call(p, ed, zer128)
        p = _layer_call(accp, p, dinv, b.reshape(1, H), w)
    accp = _agg_call(p, ed, zer128)

    batch2 = batch.reshape(NBLK, 1, RB)
    return _final_call(accp, p, dinv, b4.reshape(1, H), batch2,
                       lin1_W, lin1_b.reshape(1, H),
                       lin2_W, lin2_b.reshape(1, C))


# speedup vs baseline: 1.0084x; 1.0084x over previous
import functools

import jax
import jax.numpy as jnp
from jax import lax
from jax.experimental import pallas as pl
from jax.experimental.pallas import tpu as pltpu
from jax.experimental.pallas import tpu_sc as plsc

N = 10000
E = 320000
D = 128
H = 128
C = 10
G = 64

NC = 2
NS = 16
NW = NC * NS
EPW = E // NW
K = 80
CH = EPW // K
NT = 5
TCH = CH // NT
NPAD = 10240
RPT = NPAD // NS

_mesh = plsc.VectorSubcoreMesh(core_axis_name="c", subcore_axis_name="s",
                               num_cores=NC, num_subcores=NS)


def _deg_body(dst_hbm, ones_hbm, zeros_hbm, deg_out, dstloc, ones_v, acc):
    c = lax.axis_index("c")
    s = lax.axis_index("s")
    wid = s * NC + c
    pltpu.sync_copy(zeros_hbm, acc.at[pl.ds(s * RPT, RPT)])
    pltpu.sync_copy(ones_hbm, ones_v)
    pltpu.sync_copy(dst_hbm.at[wid], dstloc)
    plsc.subcore_barrier()

    def chunk(g, carry):
        pltpu.sync_copy(ones_v, acc.at[dstloc.at[g]], add=True)
        return carry

    lax.fori_loop(0, CH, chunk, 0)
    plsc.subcore_barrier()
    pltpu.sync_copy(acc.at[pl.ds(s * RPT, RPT)],
                    deg_out.at[c, pl.ds(s * RPT, RPT)])


_deg_call = pl.kernel(
    _deg_body,
    out_type=jax.ShapeDtypeStruct((NC, NPAD, H), jnp.float32),
    mesh=_mesh,
    scratch_types=[
        pltpu.VMEM((CH, K), jnp.int32),
        pltpu.VMEM((K, H), jnp.float32),
        pltpu.VMEM_SHARED((NPAD, H), jnp.float32),
    ],
)


NBUF = 3
TAILR = N - (NS - 1) * RPT


def _agg_body(p_hbm, ed_hbm, zeros_hbm, out_hbm,
              ed0, ed1, rows0, rows1, rows2, acc,
              seme0, seme1, semg0, semg1, semg2,
              sems0, sems1, sems2):
    c = lax.axis_index("c")
    s = lax.axis_index("s")
    wid = s * NC + c

    @pl.when(s < NS - 1)
    def _():
        pltpu.sync_copy(zeros_hbm, acc.at[pl.ds(s * RPT, RPT)])

    @pl.when(s == NS - 1)
    def _():
        pltpu.sync_copy(zeros_hbm.at[pl.ds(0, TAILR)],
                        acc.at[pl.ds((NS - 1) * RPT, TAILR)])

    eds = (ed0, ed1)
    seme = (seme0, seme1)
    rows = (rows0, rows1, rows2)
    semg = (semg0, semg1, semg2)
    sems = (sems0, sems1, sems2)
    pltpu.async_copy(ed_hbm.at[wid, 0], ed0, seme0)
    plsc.subcore_barrier()

    for t in range(NT):
        eb = eds[t % 2]
        pltpu.make_async_copy(ed_hbm.at[wid, t], eb, seme[t % 2]).wait()
        if t + 1 < NT:
            pltpu.async_copy(ed_hbm.at[wid, t + 1],
                             eds[(t + 1) % 2], seme[(t + 1) % 2])
        pltpu.async_copy(p_hbm.at[eb.at[0, 0]], rows[0], semg[0])
        pltpu.async_copy(p_hbm.at[eb.at[0, 1]], rows[1], semg[1])

        def step(i, carry, eb=eb):
            for b in range(NBUF):
                j = NBUF * i + b
                nb = (b + NBUF - 1) % NBUF

                @pl.when(j + NBUF - 1 < TCH)
                def _():
                    @pl.when(j >= 1)
                    def _():
                        pltpu.make_async_copy(
                            rows[nb], acc.at[eb.at[1, 0]], sems[nb]).wait()
                    pltpu.async_copy(p_hbm.at[eb.at[0, j + NBUF - 1]],
                                     rows[nb], semg[nb])

                pltpu.make_async_copy(p_hbm.at[eb.at[0, j]],
                                      rows[b], semg[b]).wait()
                pltpu.async_copy(rows[b], acc.at[eb.at[1, j]],
                                 sems[b], add=True)
            return carry

        lax.fori_loop(0, TCH // NBUF, step, 0)
        j = TCH - 1
        pltpu.make_async_copy(p_hbm.at[eb.at[0, j]],
                              rows[j % NBUF], semg[j % NBUF]).wait()
        pltpu.async_copy(rows[j % NBUF], acc.at[eb.at[1, j]],
                         sems[j % NBUF], add=True)
        for b in range(NBUF):
            pltpu.make_async_copy(rows[b], acc.at[eb.at[1, 0]],
                                  sems[b]).wait()

    plsc.subcore_barrier()

    @pl.when(s < NS - 1)
    def _():
        pltpu.sync_copy(acc.at[pl.ds(s * RPT, RPT)],
                        out_hbm.at[c, pl.ds(s * RPT, RPT)])

    @pl.when(s == NS - 1)
    def _():
        pltpu.sync_copy(acc.at[pl.ds((NS - 1) * RPT, TAILR)],
                        out_hbm.at[c, pl.ds((NS - 1) * RPT, TAILR)])


_agg_call = pl.kernel(
    _agg_body,
    out_type=jax.ShapeDtypeStruct((NC, N, H), jnp.float32),
    mesh=_mesh,
    scratch_types=[
        pltpu.VMEM((2, TCH, K), jnp.int32),
        pltpu.VMEM((2, TCH, K), jnp.int32),
        pltpu.VMEM((K, H), jnp.float32),
        pltpu.VMEM((K, H), jnp.float32),
        pltpu.VMEM((K, H), jnp.float32),
        pltpu.VMEM_SHARED((N, H), jnp.float32),
        pltpu.SemaphoreType.DMA,
        pltpu.SemaphoreType.DMA,
        pltpu.SemaphoreType.DMA,
        pltpu.SemaphoreType.DMA,
        pltpu.SemaphoreType.DMA,
        pltpu.SemaphoreType.DMA,
        pltpu.SemaphoreType.DMA,
        pltpu.SemaphoreType.DMA,
    ],
)


RB = 2000
NBLK = N // RB


def _prep_body(d0_ref, d1_ref, x_ref, w_ref, p_ref, dinv_ref):
    deg = d0_ref[:, 0:1] + d1_ref[:, 0:1] + 1.0
    dinv = lax.rsqrt(deg)
    dinv_ref[...] = dinv
    p_ref[...] = jnp.dot(x_ref[...] * dinv, w_ref[...],
                         preferred_element_type=jnp.float32)


def _prep_call(d0, d1, x, w):
    return pl.pallas_call(
        _prep_body,
        grid=(NBLK,),
        in_specs=[
            pl.BlockSpec((RB, H), lambda r: (r, 0)),
            pl.BlockSpec((RB, H), lambda r: (r, 0)),
            pl.BlockSpec((RB, D), lambda r: (r, 0)),
            pl.BlockSpec((D, H), lambda r: (0, 0)),
        ],
        out_specs=[
            pl.BlockSpec((RB, H), lambda r: (r, 0)),
            pl.BlockSpec((RB, 1), lambda r: (r, 0)),
        ],
        out_shape=[
            jax.ShapeDtypeStruct((N, H), jnp.float32),
            jax.ShapeDtypeStruct((N, 1), jnp.float32),
        ],
    )(d0, d1, x, w)


def _layer_body(acc_ref, p_ref, dinv_ref, b_ref, w_ref, pn_ref):
    acc = acc_ref[0] + acc_ref[1]
    dinv = dinv_ref[...]
    a = jnp.maximum(dinv * (acc + p_ref[...]) + b_ref[...], 0.0)
    pn_ref[...] = jnp.dot(a * dinv, w_ref[...],
                          preferred_element_type=jnp.float32)


def _layer_call(accp, p, dinv, b, w):
    return pl.pallas_call(
        _layer_body,
        grid=(NBLK,),
        in_specs=[
            pl.BlockSpec((NC, RB, H), lambda r: (0, r, 0)),
            pl.BlockSpec((RB, H), lambda r: (r, 0)),
            pl.BlockSpec((RB, 1), lambda r: (r, 0)),
            pl.BlockSpec((1, H), lambda r: (0, 0)),
            pl.BlockSpec((H, H), lambda r: (0, 0)),
        ],
        out_specs=pl.BlockSpec((RB, H), lambda r: (r, 0)),
        out_shape=jax.ShapeDtypeStruct((N, H), jnp.float32),
    )(accp, p, dinv, b, w)


def _final_body(acc_ref, p_ref, dinv_ref, b_ref, batch_ref,
                l1w_ref, l1b_ref, l2w_ref, l2b_ref, out_ref, psum, cnt):
    r = pl.program_id(0)

    @pl.when(r == 0)
    def _():
        psum[...] = jnp.zeros_like(psum)
        cnt[...] = jnp.zeros_like(cnt)

    acc = acc_ref[0] + acc_ref[1]
    h = jnp.maximum(dinv_ref[...] * (acc + p_ref[...]) + b_ref[...], 0.0)
    gids = lax.broadcasted_iota(jnp.int32, (G, RB), 0)
    onehot = (gids == batch_ref[0]).astype(jnp.float32)
    psum[...] += jnp.dot(onehot, h, preferred_element_type=jnp.float32)
    cnt[...] += jnp.sum(onehot, axis=1, keepdims=True)

    @pl.when(r == NBLK - 1)
    def _():
        pooled = psum[...] / jnp.maximum(cnt[...], 1.0)
        z = jnp.maximum(
            jnp.dot(pooled, l1w_ref[...], preferred_element_type=jnp.float32)
            + l1b_ref[...], 0.0)
        logits = (jnp.dot(z, l2w_ref[...], preferred_element_type=jnp.float32)
                  + l2b_ref[...])
        m = jnp.max(logits, axis=1, keepdims=True)
        lse = jnp.log(jnp.sum(jnp.exp(logits - m), axis=1, keepdims=True)) + m
        out_ref[...] = logits - lse


def _final_call(accp, p, dinv, b, batch2, l1w, l1b, l2w, l2b):
    return pl.pallas_call(
        _final_body,
        grid=(NBLK,),
        in_specs=[
            pl.BlockSpec((NC, RB, H), lambda r: (0, r, 0)),
            pl.BlockSpec((RB, H), lambda r: (r, 0)),
            pl.BlockSpec((RB, 1), lambda r: (r, 0)),
            pl.BlockSpec((1, H), lambda r: (0, 0)),
            pl.BlockSpec((1, 1, RB), lambda r: (r, 0, 0)),
            pl.BlockSpec((H, H), lambda r: (0, 0)),
            pl.BlockSpec((1, H), lambda r: (0, 0)),
            pl.BlockSpec((H, C), lambda r: (0, 0)),
            pl.BlockSpec((1, C), lambda r: (0, 0)),
        ],
        out_specs=pl.BlockSpec((G, C), lambda r: (0, 0)),
        out_shape=jax.ShapeDtypeStruct((G, C), jnp.float32),
        scratch_shapes=[
            pltpu.VMEM((G, H), jnp.float32),
            pltpu.VMEM((G, 1), jnp.float32),
        ],
    )(accp, p, dinv, b, batch2, l1w, l1b, l2w, l2b)


def kernel(x, edge_index, batch, W1, b1, W2, b2, W3, b3, W4, b4,
           lin1_W, lin1_b, lin2_W, lin2_b):
    src2 = edge_index[0].reshape(NW, CH, K)
    dst2 = edge_index[1].reshape(NW, CH, K)
    ed = jnp.stack([edge_index[0].reshape(NW, NT, TCH, K),
                    edge_index[1].reshape(NW, NT, TCH, K)], axis=2)
    ones128 = jnp.ones((K, H), jnp.float32)
    zer128 = jnp.zeros((RPT, H), jnp.float32)

    degp = _deg_call(dst2, ones128, zer128)

    p, dinv = _prep_call(degp[0], degp[1], x, W1)
    for b, w in ((b1, W2), (b2, W3), (b3, W4)):
        accp = _agg_call(p, ed, zer128)
        p = _layer_call(accp, p, dinv, b.reshape(1, H), w)
    accp = _agg_call(p, ed, zer128)

    batch2 = batch.reshape(NBLK, 1, RB)
    return _final_call(accp, p, dinv, b4.reshape(1, H), batch2,
                       lin1_W, lin1_b.reshape(1, H),
                       lin2_W, lin2_b.reshape(1, C))
